# baseline (speedup 1.0000x reference)
import math

import jax
import jax.numpy as jnp
from jax import lax
from jax.experimental import pallas as pl
from jax.experimental.pallas import tpu as pltpu
from jax.experimental.pallas import tpu_sc as plsc

N = 50000
E = 200000
D = 128
H = 64
HW = 32
O = 16
_BIAS = math.log(math.exp(1.0) - 1.0)

NC = 2
NS = 16
PT = 3128
PT_LAST = N - (NS - 1) * PT
ACC_ROWS = NS * PT
CH = 112
E_PAD = 200704
NCH = E_PAD // CH
CPT = NCH // NS
HSLAB = CPT // 2
RING = 4


def _init_acc(q128, acc, s):
    c = lax.axis_index("c")
    r0 = s * PT

    def at(col0):
        @pl.when(s < NS - 1)
        def _():
            pltpu.sync_copy(q128.at[pl.ds(r0, PT), pl.ds(col0, HW)],
                            acc.at[pl.ds(r0, PT)])

        @pl.when(s == NS - 1)
        def _():
            pltpu.sync_copy(q128.at[pl.ds(r0, PT_LAST), pl.ds(col0, HW)],
                            acc.at[pl.ds(r0, PT_LAST)])
            pltpu.sync_copy(q128.at[pl.ds(0, PT - PT_LAST), pl.ds(col0, HW)],
                            acc.at[pl.ds(r0 + PT_LAST, PT - PT_LAST)])

    @pl.when(c == 0)
    def _():
        at(2 * HW)

    @pl.when(c == 1)
    def _():
        at(3 * HW)


def _accumulate(gsrc, s2d, d2d, acc, srcb, dstb, rows, gs, ss, c_lo):
    def fire_g(k, p):
        pltpu.async_copy(gsrc.at[srcb.at[k]], rows.at[p], gs.at[p])

    def drain_g(k, p):
        pltpu.make_async_copy(gsrc.at[srcb.at[k]], rows.at[p], gs.at[p]).wait()

    def fire_s(k, p):
        pltpu.async_copy(rows.at[p], acc.at[dstb.at[k]], ss.at[p], add=True)

    def drain_s(k, p):
        pltpu.make_async_copy(rows.at[p], acc.at[dstb.at[k]], ss.at[p]).wait()

    def body(k4, carry):
        for p in range(RING):
            k = RING * k4 + p
            drain_g(k, p)
            fire_s(k, p)

            @pl.when(k + RING < HSLAB)
            def _():
                drain_s(k, p)
                fire_g(k + RING, p)

        return carry

    for h in range(CPT // HSLAB):
        pltpu.sync_copy(s2d.at[pl.ds(c_lo + h * HSLAB, HSLAB)], srcb)
        pltpu.sync_copy(d2d.at[pl.ds(c_lo + h * HSLAB, HSLAB)], dstb)
        for p in range(RING):
            fire_g(p, p)
        lax.fori_loop(0, HSLAB // RING, body, 0)
        for p in range(RING):
            drain_s(HSLAB - RING + p, p)


def _dump(out128, acc, s):
    c = lax.axis_index("c")
    r0 = s * PT

    def to(col0, n):
        pltpu.sync_copy(acc.at[pl.ds(r0, n)],
                        out128.at[pl.ds(r0, n), pl.ds(col0, HW)])

    @pl.when((c == 0) & (s < NS - 1))
    def _():
        to(0, PT)

    @pl.when((c == 0) & (s == NS - 1))
    def _():
        to(0, PT_LAST)

    @pl.when((c == 1) & (s < NS - 1))
    def _():
        to(HW, PT)

    @pl.when((c == 1) & (s == NS - 1))
    def _():
        to(HW, PT_LAST)


def _conv_pass(gview, s_lo, s_hi, d2d, acc, srcb, dstb, rows, gs, ss, c_lo):
    c = lax.axis_index("c")

    @pl.when(c == 0)
    def _():
        _accumulate(gview, s_lo, d2d, acc, srcb, dstb, rows, gs, ss, c_lo)

    @pl.when(c == 1)
    def _():
        _accumulate(gview, s_hi, d2d, acc, srcb, dstb, rows, gs, ss, c_lo)


def _sc_mt_body(vpj, q128, sl, sh, d2d, mt128,
                acc, srcb, dstb, rows, gs, ss):
    s = lax.axis_index("s")
    c_lo = s * CPT
    _init_acc(q128, acc, s)
    plsc.subcore_barrier()
    _conv_pass(vpj, sl, sh, d2d, acc, srcb, dstb, rows, gs, ss, c_lo)
    plsc.subcore_barrier()
    _dump(mt128, acc, s)


def _sc_mj_body(vat, vbj, q128, sl_tj, sh_tj, d_tj, sl_jj, sh_jj, d_jj,
                mj128, acc, srcb, dstb, rows, gs, ss):
    s = lax.axis_index("s")
    c_lo = s * CPT
    _init_acc(q128, acc, s)
    plsc.subcore_barrier()
    _conv_pass(vat, sl_tj, sh_tj, d_tj, acc, srcb, dstb, rows, gs, ss, c_lo)
    _conv_pass(vbj, sl_jj, sh_jj, d_jj, acc, srcb, dstb, rows, gs, ss, c_lo)
    plsc.subcore_barrier()
    _dump(mj128, acc, s)


_SC_SCRATCH = (
    pltpu.VMEM_SHARED((ACC_ROWS, HW), jnp.float32),
    pltpu.VMEM((HSLAB, CH), jnp.int32),
    pltpu.VMEM((HSLAB, CH), jnp.int32),
    pltpu.VMEM((RING, CH, HW), jnp.float32),
    pltpu.SemaphoreType.DMA((RING,)),
    pltpu.SemaphoreType.DMA((RING,)),
)
_MESH = plsc.VectorSubcoreMesh(core_axis_name="c", subcore_axis_name="s")
_OUT128 = jax.ShapeDtypeStruct((N, 4 * HW), jnp.float32)
_sc_mt = pl.kernel(
    _sc_mt_body, out_type=(_OUT128,), mesh=_MESH, scratch_types=_SC_SCRATCH,
    compiler_params=pltpu.CompilerParams(use_tc_tiling_on_sc=False))
_sc_mj = pl.kernel(
    _sc_mj_body, out_type=(_OUT128,), mesh=_MESH, scratch_types=_SC_SCRATCH,
    compiler_params=pltpu.CompilerParams(use_tc_tiling_on_sc=False))


R = 2000
_P = jax.lax.Precision.DEFAULT


def _dot(a, b):
    return jnp.dot(a, b, precision=_P, preferred_element_type=jnp.float32)


def _cat(a, b):
    return jnp.concatenate([a, b], axis=1)


def _f1_body(xt, xj, wit, bit, wij, bij, wr_tj, wr_jt, wr_jj,
             wq_t, bq_t, wq_j, bq_j, at_out, pj_out, bj_out):
    ht = _dot(xt[...], wit[...]) + bit[...]
    hj = _dot(xj[...], wij[...]) + bij[...]
    at_out[...] = _cat(_dot(ht, wr_tj[...]), _dot(ht, wq_t[...]) + bq_t[...])
    pj_out[...] = _cat(_dot(hj, wr_jt[...]), _dot(hj, wq_j[...]) + bq_j[...])
    bj_out[...] = _cat(_dot(hj, wr_jj[...]), jnp.zeros((R, H), jnp.float32))


def _f2_t_body(mt, wr_tj, wq_t, bq_t, at_out):
    ht = jnp.tanh(mt[:, :H])
    at_out[...] = _cat(_dot(ht, wr_tj[...]), _dot(ht, wq_t[...]) + bq_t[...])


def _f2_j_body(mj, wr_jt, wr_jj, wq_j, bq_j, pj_out, bj_out):
    hj = jnp.tanh(mj[:, :H])
    pj_out[...] = _cat(_dot(hj, wr_jt[...]), _dot(hj, wq_j[...]) + bq_j[...])
    bj_out[...] = _cat(_dot(hj, wr_jj[...]), jnp.zeros((R, H), jnp.float32))


def _f3_body(m, wo, bo, y_out):
    h = jnp.tanh(m[:, :H])
    y = jnp.tanh(_dot(h, wo[...]) + bo[...])
    loc = y[:, :O // 2]
    v = y[:, O // 2:] + _BIAS
    sp = jnp.log1p(jnp.exp(-jnp.abs(v))) + jnp.maximum(v, 0.0)
    scale = jnp.maximum(sp, 1e-4)
    y_out[...] = jnp.concatenate(
        [loc, scale, jnp.zeros((R, 4 * HW - O), jnp.float32)], axis=1)


def _spec(rows, cols):
    return pl.BlockSpec((rows, cols), lambda i: (i, 0))


def _w_spec(r, cc):
    return pl.BlockSpec((r, cc), lambda i: (0, 0))


def _call(body, in_rc, w_shapes, n_out):
    grid = N // R
    in_specs = [_spec(*rc) for rc in in_rc] + [_w_spec(*sh) for sh in w_shapes]
    return pl.pallas_call(
        body,
        grid=(grid,),
        in_specs=in_specs,
        out_specs=[_spec(R, 4 * HW)] * n_out,
        out_shape=[jax.ShapeDtypeStruct((N, 4 * HW), jnp.float32)] * n_out,
    )


_P128 = (R, 4 * HW)
_WH = (H, H)
_B = (1, H)
_f1 = _call(_f1_body, [(R, D), (R, D)],
            [(D, H), _B, (D, H), _B, _WH, _WH, _WH, _WH, _B, _WH, _B], 3)
_f2_t = _call(_f2_t_body, [_P128], [_WH, _WH, _B], 1)
_f2_j = _call(_f2_j_body, [_P128], [_WH, _WH, _WH, _B], 2)
_f3 = _call(_f3_body, [_P128], [(H, O), (1, O)], 1)


def kernel(x_torso, x_joint, edge_index_tj, edge_index_jt, edge_index_jj,
           Wi_t, bi_t, Wi_j, bi_j,
           Wr1_tj, br1_tj, Wq1_tj, Wr1_jt, br1_jt, Wq1_jt, Wr1_jj, br1_jj, Wq1_jj,
           Wr2_tj, br2_tj, Wq2_tj, Wr2_jt, br2_jt, Wq2_jt, Wr2_jj, br2_jj, Wq2_jj,
           Wo_t, bo_t, Wo_j, bo_j):
    pad_src = (jnp.arange(E_PAD - E, dtype=jnp.int32) * 41) % N
    pad_dst = N + (jnp.arange(E_PAD - E, dtype=jnp.int32) & 7)

    def _prep(ei):
        src = jnp.concatenate([ei[0], pad_src])
        dst = jnp.concatenate([ei[1], pad_dst])
        s4 = 4 * src
        return (s4.reshape(NCH, CH), (s4 + 1).reshape(NCH, CH),
                dst.reshape(NCH, CH))

    sl_tj, sh_tj, d_tj = _prep(edge_index_tj)
    sl_jt, sh_jt, d_jt = _prep(edge_index_jt)
    sl_jj, sh_jj, d_jj = _prep(edge_index_jj)
    r2 = lambda b: b.reshape(1, -1)
    view = lambda p: p.reshape(4 * N, HW)
    wq1_j = Wq1_tj + Wq1_jj
    bq1_j = r2(br1_tj + br1_jj)
    wq2_j = Wq2_tj + Wq2_jj
    bq2_j = r2(br2_tj + br2_jj)

    at1, pj1, bj1 = _f1(x_torso, x_joint, Wi_t, r2(bi_t), Wi_j, r2(bi_j),
                        Wr1_tj, Wr1_jt, Wr1_jj, Wq1_jt, r2(br1_jt),
                        wq1_j, bq1_j)
    (mt1,) = _sc_mt(view(pj1), at1, sl_jt, sh_jt, d_jt)
    (mj1,) = _sc_mj(view(at1), view(bj1), pj1,
                    sl_tj, sh_tj, d_tj, sl_jj, sh_jj, d_jj)
    (at2,) = _f2_t(mt1, Wr2_tj, Wq2_jt, r2(br2_jt))
    pj2, bj2 = _f2_j(mj1, Wr2_jt, Wr2_jj, wq2_j, bq2_j)
    (mt2,) = _sc_mt(view(pj2), at2, sl_jt, sh_jt, d_jt)
    (mj2,) = _sc_mj(view(at2), view(bj2), pj2,
                    sl_tj, sh_tj, d_tj, sl_jj, sh_jj, d_jj)
    (yt,) = _f3(mt2, Wo_t, r2(bo_t))
    (yj,) = _f3(mj2, Wo_j, r2(bo_j))
    half = O // 2
    return (yt[:, :half], yt[:, half:O], yj[:, :half], yj[:, half:O])
